# baseline (speedup 1.0000x reference)
import functools

import jax
import jax.numpy as jnp
from jax import lax
from jax.experimental import pallas as pl
from jax.experimental.pallas import tpu as pltpu
from jax.experimental.pallas import tpu_sc as plsc

_L = 16
_NW = 32
_U = 128
_NE = 119


def _n_pad(n: int) -> int:
    units = -(-n // _U)
    return -(-units // _NW) * _NW * _U


def _make_sc_kernel(n: int):
    n_pad = _n_pad(n)
    ch = n_pad // _NW
    last_base = (_NW - 1) * ch
    z_full = ((n - last_base) // _U) * _U
    z_rem = n - last_base - z_full
    assert z_rem % 8 == 0

    mesh = plsc.VectorSubcoreMesh(core_axis_name="c", subcore_axis_name="s")

    @functools.partial(
        pl.kernel,
        out_type=jax.ShapeDtypeStruct((3, n_pad), jnp.float32),
        mesh=mesh,
        scratch_types=[
            pltpu.VMEM((ch,), jnp.int32),
            pltpu.VMEM((3, ch), jnp.float32),
            pltpu.VMEM((_NE,), jnp.float32),
            pltpu.VMEM((_NE,), jnp.float32),
            pltpu.SemaphoreType.DMA,
        ],
        compiler_params=pltpu.CompilerParams(needs_layout_passes=False),
    )
    def body(f_hbm, z_hbm, eta_hbm, out_hbm, z_v, f_v, eta_v, inv_v, sem_f):
        wid = lax.axis_index("s") * 2 + lax.axis_index("c")
        base = wid * ch

        f_in = pltpu.async_copy(f_hbm.at[:, pl.ds(base, ch)], f_v, sem_f)

        @pl.when(wid < _NW - 1)
        def _():
            pltpu.sync_copy(z_hbm.at[pl.ds(base, ch)], z_v)

        @pl.when(wid == _NW - 1)
        def _():
            pltpu.sync_copy(z_hbm.at[pl.ds(base, z_full)],
                            z_v.at[pl.ds(0, z_full)])
            if z_rem:
                pltpu.sync_copy(z_hbm.at[pl.ds(base + z_full, z_rem)],
                                z_v.at[pl.ds(z_full, z_rem)])

        pltpu.sync_copy(eta_hbm, eta_v)
        for i in range(8):
            sl = pl.ds(min(i * _L, _NE - _L), _L)
            inv_v[sl] = 1.0 / eta_v[sl]

        f_in.wait()

        def group(g, carry):
            sl = pl.ds(g * _L, _L)
            r = plsc.load_gather(inv_v, [jnp.minimum(z_v[sl], _NE - 1)])
            for c in range(3):
                f_v[c, sl] = f_v[c, sl] * r
            return carry

        lax.fori_loop(0, ch // _L, group, 0, unroll=16)

        pltpu.sync_copy(f_v, out_hbm.at[:, pl.ds(base, ch)])

    return body


def kernel(forces, Z, eta):
    n = forces.shape[0]
    ft = jnp.pad(forces.T, ((0, 0), (0, _n_pad(n) - n)))
    out = _make_sc_kernel(n)(ft, Z.astype(jnp.int32), eta)
    return out[:, :n].T


# speedup vs baseline: 1.0158x; 1.0158x over previous
import functools

import jax
import jax.numpy as jnp
from jax import lax
from jax.experimental import pallas as pl
from jax.experimental.pallas import tpu as pltpu
from jax.experimental.pallas import tpu_sc as plsc

_L = 16
_NW = 32
_U = 128
_NE = 119


def _n_pad(n: int) -> int:
    units = -(-n // _U)
    return -(-units // _NW) * _NW * _U


def _make_sc_kernel(n: int):
    n_pad = _n_pad(n)
    ch = n_pad // _NW
    last_base = (_NW - 1) * ch
    z_full = ((n - last_base) // _U) * _U
    z_rem = n - last_base - z_full
    assert z_rem % 8 == 0

    mesh = plsc.VectorSubcoreMesh(core_axis_name="c", subcore_axis_name="s")

    @functools.partial(
        pl.kernel,
        out_type=jax.ShapeDtypeStruct((3, n_pad), jnp.float32),
        mesh=mesh,
        scratch_types=[
            pltpu.VMEM((ch,), jnp.int32),
            pltpu.VMEM((3, ch), jnp.float32),
            pltpu.VMEM((_NE,), jnp.float32),
            pltpu.VMEM((_NE,), jnp.float32),
            pltpu.SemaphoreType.DMA,
            pltpu.SemaphoreType.DMA,
            pltpu.SemaphoreType.DMA,
            pltpu.SemaphoreType.DMA,
        ],
        compiler_params=pltpu.CompilerParams(needs_layout_passes=False),
    )
    def body(f_hbm, z_hbm, eta_hbm, out_hbm, z_v, f_v, eta_v, inv_v,
             sem_a, sem_b, sem_c, sem_d):
        wid = lax.axis_index("s") * 2 + lax.axis_index("c")
        base = wid * ch
        half = ((ch // 2 + _U - 1) // _U) * _U
        rest = ch - half

        in0 = pltpu.async_copy(f_hbm.at[:, pl.ds(base, half)],
                               f_v.at[:, pl.ds(0, half)], sem_a)
        in1 = pltpu.async_copy(f_hbm.at[:, pl.ds(base + half, rest)],
                               f_v.at[:, pl.ds(half, rest)], sem_b)

        @pl.when(wid < _NW - 1)
        def _():
            pltpu.sync_copy(z_hbm.at[pl.ds(base, ch)], z_v)

        @pl.when(wid == _NW - 1)
        def _():
            pltpu.sync_copy(z_hbm.at[pl.ds(base, z_full)],
                            z_v.at[pl.ds(0, z_full)])
            if z_rem:
                pltpu.sync_copy(z_hbm.at[pl.ds(base + z_full, z_rem)],
                                z_v.at[pl.ds(z_full, z_rem)])

        pltpu.sync_copy(eta_hbm, eta_v)
        for i in range(8):
            sl = pl.ds(min(i * _L, _NE - _L), _L)
            inv_v[sl] = 1.0 / eta_v[sl]

        def group(g, carry):
            sl = pl.ds(g * _L, _L)
            r = plsc.load_gather(inv_v, [jnp.minimum(z_v[sl], _NE - 1)])
            for c in range(3):
                f_v[c, sl] = f_v[c, sl] * r
            return carry

        in0.wait()
        lax.fori_loop(0, half // _L, group, 0, unroll=8)
        out0 = pltpu.async_copy(f_v.at[:, pl.ds(0, half)],
                                out_hbm.at[:, pl.ds(base, half)], sem_c)
        in1.wait()
        lax.fori_loop(half // _L, ch // _L, group, 0, unroll=8)
        out1 = pltpu.async_copy(f_v.at[:, pl.ds(half, rest)],
                                out_hbm.at[:, pl.ds(base + half, rest)], sem_d)
        out0.wait()
        out1.wait()

    return body


def kernel(forces, Z, eta):
    n = forces.shape[0]
    ft = jnp.pad(forces.T, ((0, 0), (0, _n_pad(n) - n)))
    out = _make_sc_kernel(n)(ft, Z.astype(jnp.int32), eta)
    return out[:, :n].T


# speedup vs baseline: 1.0853x; 1.0685x over previous
import functools

import jax
import jax.numpy as jnp
from jax import lax
from jax.experimental import pallas as pl
from jax.experimental.pallas import tpu as pltpu
from jax.experimental.pallas import tpu_sc as plsc

_L = 16
_NW = 32
_U = 128
_NE = 119


def _n_pad(n: int) -> int:
    units = -(-n // _U)
    return -(-units // _NW) * _NW * _U


def _make_sc_kernel(n: int):
    n_pad = _n_pad(n)
    ch = n_pad // _NW
    last_base = (_NW - 1) * ch
    z_full = ((n - last_base) // _U) * _U
    z_rem = n - last_base - z_full
    assert z_rem % 8 == 0

    mesh = plsc.VectorSubcoreMesh(core_axis_name="c", subcore_axis_name="s")

    @functools.partial(
        pl.kernel,
        out_type=jax.ShapeDtypeStruct((3, n_pad), jnp.float32),
        mesh=mesh,
        scratch_types=[
            pltpu.VMEM((ch,), jnp.int32),
            pltpu.VMEM((3, ch), jnp.float32),
            pltpu.VMEM((_NE,), jnp.float32),
            pltpu.VMEM((_NE,), jnp.float32),
            pltpu.SemaphoreType.DMA,
            pltpu.SemaphoreType.DMA,
            pltpu.SemaphoreType.DMA,
            pltpu.SemaphoreType.DMA,
        ],
        compiler_params=pltpu.CompilerParams(needs_layout_passes=False),
    )
    def body(f_hbm, z_hbm, eta_hbm, out_hbm, z_v, f_v, eta_v, inv_v,
             sem_a, sem_b, sem_c, sem_d):
        wid = lax.axis_index("s") * 2 + lax.axis_index("c")
        base = wid * ch
        half = ((ch // 2 + _U - 1) // _U) * _U
        rest = ch - half

        in0 = pltpu.async_copy(f_hbm.at[:, pl.ds(base, half)],
                               f_v.at[:, pl.ds(0, half)], sem_a)
        in1 = pltpu.async_copy(f_hbm.at[:, pl.ds(base + half, rest)],
                               f_v.at[:, pl.ds(half, rest)], sem_b)

        @pl.when(wid < _NW - 1)
        def _():
            pltpu.sync_copy(z_hbm.at[pl.ds(base, ch)], z_v)

        @pl.when(wid == _NW - 1)
        def _():
            pltpu.sync_copy(z_hbm.at[pl.ds(base, z_full)],
                            z_v.at[pl.ds(0, z_full)])
            if z_rem:
                pltpu.sync_copy(z_hbm.at[pl.ds(base + z_full, z_rem)],
                                z_v.at[pl.ds(z_full, z_rem)])

        pltpu.sync_copy(eta_hbm, eta_v)
        for i in range(8):
            sl = pl.ds(min(i * _L, _NE - _L), _L)
            inv_v[sl] = 1.0 / eta_v[sl]

        def group(g):
            sl = pl.ds(g * _L, _L)
            r = plsc.load_gather(inv_v, [jnp.minimum(z_v[sl], _NE - 1)])
            for c in range(3):
                f_v[c, sl] = f_v[c, sl] * r

        in0.wait()
        plsc.parallel_loop(0, half // _L, unroll=8)(group)
        out0 = pltpu.async_copy(f_v.at[:, pl.ds(0, half)],
                                out_hbm.at[:, pl.ds(base, half)], sem_c)
        in1.wait()
        plsc.parallel_loop(half // _L, ch // _L, unroll=8)(group)
        out1 = pltpu.async_copy(f_v.at[:, pl.ds(half, rest)],
                                out_hbm.at[:, pl.ds(base + half, rest)], sem_d)
        out0.wait()
        out1.wait()

    return body


def kernel(forces, Z, eta):
    n = forces.shape[0]
    ft = jnp.pad(forces.T, ((0, 0), (0, _n_pad(n) - n)))
    out = _make_sc_kernel(n)(ft, Z.astype(jnp.int32), eta)
    return out[:, :n].T
